# baseline (speedup 1.0000x reference)
import functools

import jax
import jax.numpy as jnp
from jax import lax
from jax.experimental import pallas as pl
from jax.experimental.pallas import tpu as pltpu
from jax.experimental.pallas import tpu_sc as plsc

_B = 4096
_S = 50
_D = 64
_V = 100000
_L = 16
_NC = 2
_NS = 16
_NW = _NC * _NS
_BPW = _B // _NW
_CB = 16
_NCHUNK = _BPW // _CB
_SP = 56
_IDXN = _CB * _SP


def _sc_body(w_hbm, qf_hbm, qlen_hbm, out_hbm,
             idx_a, idx_b, rows_a, rows_b, qlen_a, qlen_b, lens_v, out_v,
             sem_a, sem_b):
  wid = lax.axis_index("s") * _NC + lax.axis_index("c")
  idx_bufs = (idx_a, idx_b)
  rows_bufs = (rows_a, rows_b)
  qlen_bufs = (qlen_a, qlen_b)
  sems = (sem_a, sem_b)

  def stage(g):
    buf = g % 2
    base = wid * _BPW + g * _CB
    pltpu.sync_copy(qf_hbm.at[pl.ds(base, _CB), pl.ds(0, _SP)], idx_bufs[buf])
    pltpu.sync_copy(qlen_hbm.at[pl.ds(base, _CB)], qlen_bufs[buf])
    descs = []
    for b in range(_CB):
      descs.append(pltpu.async_copy(
          w_hbm.at[idx_bufs[buf].at[b]],
          rows_bufs[buf].at[pl.ds(b * _SP, _SP)],
          sems[buf]))
    return descs

  descs = stage(0)
  for g in range(_NCHUNK):
    buf = g % 2
    next_descs = stage(g + 1) if g + 1 < _NCHUNK else None
    for d in descs:
      d.wait()
    descs = next_descs

    rows = rows_bufs[buf]
    qlen_f = qlen_bufs[buf][...].astype(jnp.float32) + 1e-12
    lens_v[pl.ds(0, _L)] = qlen_f
    lens_v[pl.ds(_L, _L)] = qlen_f

    def batch_body(b, carry):
      rv = jnp.full((_L,), lens_v[pl.ds(b, _L)][0], jnp.float32)
      row0 = b * _SP

      def seq_body(s, accs):
        i = row0 + s
        return tuple(accs[c] + rows[i, pl.ds(c * _L, _L)] for c in range(4))

      z = jnp.zeros((_L,), jnp.float32)
      accs = lax.fori_loop(0, _S, seq_body, (z, z, z, z), unroll=5)
      for c in range(4):
        out_v[b, pl.ds(c * _L, _L)] = accs[c] / rv
      return carry

    lax.fori_loop(0, _CB, batch_body, 0)
    pltpu.sync_copy(out_v, out_hbm.at[pl.ds(wid * _BPW + g * _CB, _CB)])


@jax.jit
def _bow_pool(q2d, q_len, w):
  mesh = plsc.VectorSubcoreMesh(core_axis_name="c", subcore_axis_name="s",
                                num_cores=_NC, num_subcores=_NS)
  run = pl.kernel(
      _sc_body,
      out_type=jax.ShapeDtypeStruct((_B, _D), jnp.float32),
      mesh=mesh,
      compiler_params=pltpu.CompilerParams(use_tc_tiling_on_sc=False),
      scratch_types=[
          pltpu.VMEM((_CB, _SP), jnp.int32),
          pltpu.VMEM((_CB, _SP), jnp.int32),
          pltpu.VMEM((_IDXN, _D), jnp.float32),
          pltpu.VMEM((_IDXN, _D), jnp.float32),
          pltpu.VMEM((_CB,), jnp.int32),
          pltpu.VMEM((_CB,), jnp.int32),
          pltpu.VMEM((2 * _CB,), jnp.float32),
          pltpu.VMEM((_CB, _D), jnp.float32),
          pltpu.SemaphoreType.DMA,
          pltpu.SemaphoreType.DMA,
      ],
  )
  qp = jnp.pad(q2d, ((0, 0), (0, 2 * _D - _S)))
  return run(w, qp, q_len)


def kernel(q, q_len, W):
  return _bow_pool(q, q_len, W)


# speedup vs baseline: 5.1741x; 5.1741x over previous
import jax
import jax.numpy as jnp
from jax import lax
from jax.experimental import pallas as pl
from jax.experimental.pallas import tpu as pltpu
from jax.experimental.pallas import tpu_sc as plsc

_B = 4096
_S = 50
_D = 64
_V = 100000
_L = 16
_NC = 2
_NS = 16
_NW = _NC * _NS
_BPW = _B // _NW
_CB = 16
_NCHUNK = _BPW // _CB
_IDXN = _CB * _S


def _sc_body(w_hbm, qf_hbm, qlen_hbm, out_hbm,
             idx_a, idx_b, rows_a, rows_b, qlen_a, qlen_b, lens_v, out_v,
             sem_a, sem_b):
  wid = lax.axis_index("s") * _NC + lax.axis_index("c")
  idx_bufs = (idx_a, idx_b)
  rows_bufs = (rows_a, rows_b)
  qlen_bufs = (qlen_a, qlen_b)
  sems = (sem_a, sem_b)

  def stage(g):
    buf = g % 2
    base = wid * _BPW + g * _CB
    pltpu.sync_copy(qf_hbm.at[pl.ds(base * _S, _IDXN)], idx_bufs[buf])
    pltpu.sync_copy(qlen_hbm.at[pl.ds(base, _CB)], qlen_bufs[buf])
    return [pltpu.async_copy(
        w_hbm.at[idx_bufs[buf]], rows_bufs[buf], sems[buf])]

  descs = stage(0)
  for g in range(_NCHUNK):
    buf = g % 2
    next_descs = stage(g + 1) if g + 1 < _NCHUNK else None
    for d in descs:
      d.wait()
    descs = next_descs

    rows = rows_bufs[buf]
    qlen_f = qlen_bufs[buf][...].astype(jnp.float32) + 1e-12
    lens_v[pl.ds(0, _L)] = qlen_f
    lens_v[pl.ds(_L, _L)] = qlen_f

    def batch_body(b, carry):
      rv = jnp.full((_L,), lens_v[pl.ds(b, _L)][0], jnp.float32)
      row0 = b * _S

      def seq_body(s, accs):
        i = row0 + s
        return tuple(accs[c] + rows[i, pl.ds(c * _L, _L)] for c in range(4))

      z = jnp.zeros((_L,), jnp.float32)
      accs = lax.fori_loop(0, _S, seq_body, (z, z, z, z), unroll=5)
      for c in range(4):
        out_v[b, pl.ds(c * _L, _L)] = accs[c] / rv
      return carry

    lax.fori_loop(0, _CB, batch_body, 0)
    pltpu.sync_copy(out_v, out_hbm.at[pl.ds(wid * _BPW + g * _CB, _CB)])


@jax.jit
def _bow_pool(q2d, q_len, w):
  mesh = plsc.VectorSubcoreMesh(core_axis_name="c", subcore_axis_name="s",
                                num_cores=_NC, num_subcores=_NS)
  run = pl.kernel(
      _sc_body,
      out_type=jax.ShapeDtypeStruct((_B, _D), jnp.float32),
      mesh=mesh,
      compiler_params=pltpu.CompilerParams(use_tc_tiling_on_sc=False),
      scratch_types=[
          pltpu.VMEM((_IDXN,), jnp.int32),
          pltpu.VMEM((_IDXN,), jnp.int32),
          pltpu.VMEM((_IDXN, _D), jnp.float32),
          pltpu.VMEM((_IDXN, _D), jnp.float32),
          pltpu.VMEM((_CB,), jnp.int32),
          pltpu.VMEM((_CB,), jnp.int32),
          pltpu.VMEM((2 * _CB,), jnp.float32),
          pltpu.VMEM((_CB, _D), jnp.float32),
          pltpu.SemaphoreType.DMA,
          pltpu.SemaphoreType.DMA,
      ],
  )
  return run(w, q2d.reshape(_B * _S), q_len)


def kernel(q, q_len, W):
  return _bow_pool(q, q_len, W)
